# baseline (speedup 1.0000x reference)
import functools

import jax
import jax.numpy as jnp
from jax import lax
from jax.experimental import pallas as pl
from jax.experimental.pallas import tpu as pltpu
from jax.experimental.pallas import tpu_sc as plsc

B = 4096
B3 = 3 * B
D = 128
K = 64
N_REL = 64
KG_L2 = 1e-05
SOCIAL_L2 = 1e-05

NW = 32
CHUNK = 128
NCH = B3 // NW // CHUNK
ROWS_W = B3 // NW
NBUF = 6

SM = 256
NSB = B // SM
RG = 8
NG = N_REL // RG


@functools.cache
def _sc_gather_fn():
  @functools.partial(
      pl.kernel,
      mesh=plsc.VectorSubcoreMesh(core_axis_name="c", subcore_axis_name="s"),
      out_type=[
          jax.ShapeDtypeStruct((B3, D), jnp.float32),
          jax.ShapeDtypeStruct((B3, D), jnp.float32),
          jax.ShapeDtypeStruct((B3, D), jnp.float32),
      ],
      scratch_types=(
          [pltpu.VMEM((NCH, CHUNK), jnp.int32)] * 2 +
          [pltpu.VMEM((CHUNK, D), jnp.float32)] * NBUF +
          [pltpu.SemaphoreType.DMA] * (2 * NBUF)
      ),
  )
  def _sc_gather(ent_idx, soc_idx, entity_t, user_t,
                 ent3_o, user3_o, entu3_o,
                 idx_e, idx_s, *bufs_sems):
    bufs = bufs_sems[:NBUF]
    gsem = bufs_sems[NBUF:2 * NBUF]
    wsem = bufs_sems[2 * NBUF:]

    wid = lax.axis_index("s") * 2 + lax.axis_index("c")
    row0 = wid * ROWS_W

    pltpu.sync_copy(ent_idx.at[wid], idx_e)
    pltpu.sync_copy(soc_idx.at[wid], idx_s)

    tasks = (
        [(idx_e.at[j], entity_t, ent3_o.at[pl.ds(row0 + j * CHUNK, CHUNK)])
         for j in range(NCH)] +
        [(idx_s.at[j], user_t, user3_o.at[pl.ds(row0 + j * CHUNK, CHUNK)])
         for j in range(NCH)] +
        [(idx_s.at[j], entity_t, entu3_o.at[pl.ds(row0 + j * CHUNK, CHUNK)])
         for j in range(NCH)]
    )
    nt = len(tasks)

    gathers = [None] * nt
    writes = [None] * nt
    for k in range(NBUF):
        idx_r, tab, _ = tasks[k]
        gathers[k] = pltpu.async_copy(tab.at[idx_r], bufs[k], gsem[k])
    for k in range(nt):
        b = k % NBUF
        gathers[k].wait()
        writes[k] = pltpu.async_copy(bufs[b], tasks[k][2], wsem[b])
        nxt = k + NBUF
        if nxt < nt:
            writes[nxt - NBUF].wait()
            idx_r, tab, _ = tasks[nxt]
            gathers[nxt] = pltpu.async_copy(tab.at[idx_r], bufs[b], gsem[b])
    for k in range(nt - NBUF, nt):
        writes[k].wait()

  return _sc_gather


def _row_sums(x):
    return jnp.dot(x, jnp.ones((x.shape[1], 1), jnp.float32),
                   preferred_element_type=jnp.float32)


def _tc_body(lohi_ref, e3_ref, tm_ref, rs_ref, rsf_ref, relt_ref,
             u3_ref, eu3_ref, out_ref, acc_ref):
    i = pl.program_id(0)

    e_blk = e3_ref[...].reshape(3 * SM, D).astype(jnp.bfloat16)
    rr = rs_ref[...]
    lo = lohi_ref[0, i]
    hi = lohi_ref[1, i]

    for sec in range(3):
        acc_ref[pl.ds(sec * B + i * SM, SM), :] = jnp.zeros(
            (SM, K), jnp.float32)

    def trip(j, carry):
        mj = tm_ref[pl.ds(j * D, D), :]
        y = jnp.dot(e_blk, mj, preferred_element_type=jnp.float32)
        for sec in range(3):
            ys = y[sec * SM:(sec + 1) * SM]
            contrib = jnp.zeros((SM, K), jnp.float32)
            for q in range(RG):
                m = (rr == j * RG + q).astype(jnp.float32)
                contrib = contrib + ys[:, q * K:(q + 1) * K] * m
            acc_ref[pl.ds(sec * B + i * SM, SM), :] += contrib
        return carry

    lax.fori_loop(lo, hi + 1, trip, 0)

    @pl.when(i == NSB - 1)
    def _():
        def norm(x):
            n = jnp.sqrt(_row_sums(x * x))
            inv = 1.0 / jnp.maximum(n, 1e-12)
            return x * inv, n * inv

        rsf = rsf_ref[...]
        oh = (rsf == lax.broadcasted_iota(jnp.int32, (B, N_REL), 1)
              ).astype(jnp.float32)
        relg = jnp.dot(oh, relt_ref[...],
                       preferred_element_type=jnp.float32)

        h_n, h_u = norm(acc_ref[pl.ds(0, B), :])
        p_n, p_u = norm(acc_ref[pl.ds(B, B), :])
        n_n, n_u = norm(acc_ref[pl.ds(2 * B, B), :])
        r_n, r_u = norm(relg)
        base = h_n + r_n
        pos_score = _row_sums(jnp.square(base - p_n))
        neg_score = _row_sums(jnp.square(base - n_n))
        x = neg_score - pos_score
        kg_loss = jnp.mean(jnp.maximum(-x, 0.0)
                           + jnp.log(1.0 + jnp.exp(-jnp.abs(x))))
        kg_l2 = 0.5 * (jnp.mean(jnp.square(h_u)) + jnp.mean(jnp.square(r_u))
                       + jnp.mean(jnp.square(p_u))
                       + jnp.mean(jnp.square(n_u)))

        inv_u = u3_ref[pl.ds(0, B), :]
        inv_s = inv_u + eu3_ref[pl.ds(0, B), :]
        vp_s = u3_ref[pl.ds(B, B), :] + eu3_ref[pl.ds(B, B), :]
        vn_s = u3_ref[pl.ds(2 * B, B), :] + eu3_ref[pl.ds(2 * B, B), :]
        pos_s = _row_sums(inv_s * vp_s)
        neg_s = _row_sums(inv_s * vn_s)
        xs = pos_s - neg_s
        sig = 1.0 / (1.0 + jnp.exp(-xs))
        social_loss = jnp.mean(-jnp.log(1e-10 + sig))
        social_l2 = 0.5 * (jnp.mean(_row_sums(inv_u * inv_u))
                           + jnp.mean(_row_sums(vp_s * vp_s))
                           + jnp.mean(_row_sums(vn_s * vn_s)))

        out_ref[0, 0] = (kg_loss + KG_L2 * kg_l2
                         + social_loss + SOCIAL_L2 * social_l2)


def _tc_call(lohi, e3s, tm, rs, relt, u3, eu3):
    return pl.pallas_call(
        _tc_body,
        grid=(NSB,),
        in_specs=[
            pl.BlockSpec(memory_space=pltpu.SMEM),
            pl.BlockSpec((3, SM, D), lambda i: (0, i, 0)),
            pl.BlockSpec((NG * D, RG * K), lambda i: (0, 0)),
            pl.BlockSpec((SM, 1), lambda i: (i, 0)),
            pl.BlockSpec((B, 1), lambda i: (0, 0)),
            pl.BlockSpec((N_REL, K), lambda i: (0, 0)),
            pl.BlockSpec((B3, D), lambda i: (0, 0)),
            pl.BlockSpec((B3, D), lambda i: (0, 0)),
        ],
        out_specs=pl.BlockSpec(memory_space=pltpu.SMEM),
        out_shape=jax.ShapeDtypeStruct((1, 1), jnp.float32),
        scratch_shapes=[pltpu.VMEM((B3, K), jnp.float32)],
        compiler_params=pltpu.CompilerParams(
            dimension_semantics=("arbitrary",)),
    )(lohi, e3s, tm, rs, rs.reshape(B, 1), relt, u3, eu3)


def kernel(inviter_ids, voter_pos_ids, voter_neg_ids, h, r, pos_t, neg_t,
           is_train, user_embed, entity_embed, relation_embed, trans_M):
    i32 = jnp.int32
    r_s, h_s, p_s, n_s = lax.sort(
        (r.astype(i32), h.astype(i32), pos_t.astype(i32), neg_t.astype(i32)),
        num_keys=1)

    ent_idx = jnp.concatenate([h_s, p_s, n_s]).reshape(NW, NCH, CHUNK)
    soc_idx = jnp.concatenate(
        [inviter_ids, voter_pos_ids, voter_neg_ids]).astype(i32).reshape(
        NW, NCH, CHUNK)

    ent3, user3, entu3 = _sc_gather_fn()(
        ent_idx, soc_idx, entity_embed, user_embed)

    g_all = r_s // RG
    lohi = jnp.stack([g_all[::SM], g_all[SM - 1::SM]]).astype(i32)

    tm = trans_M.reshape(NG, RG, D, K).transpose(0, 2, 1, 3).reshape(
        NG * D, RG * K).astype(jnp.bfloat16)

    out = _tc_call(lohi, ent3.reshape(3, B, D), tm,
                   r_s.reshape(B, 1), relation_embed, user3, entu3)
    return out.reshape(())


# speedup vs baseline: 1.0127x; 1.0127x over previous
import functools

import jax
import jax.numpy as jnp
from jax import lax
from jax.experimental import pallas as pl
from jax.experimental.pallas import tpu as pltpu
from jax.experimental.pallas import tpu_sc as plsc

B = 4096
B3 = 3 * B
D = 128
K = 64
N_REL = 64
KG_L2 = 1e-05
SOCIAL_L2 = 1e-05

NW = 32
CHUNK = 128
NCH = B3 // NW // CHUNK
ROWS_W = B3 // NW
NBUF = 6

SM = 256
NSB = B // SM
RG = 4
NG = N_REL // RG


def _ring_gather(tasks, bufs, gsem, wsem):
    nb = len(bufs)
    nt = len(tasks)
    gathers = [None] * nt
    writes = [None] * nt
    for k in range(min(nb, nt)):
        idx_r, tab, _ = tasks[k]
        gathers[k] = pltpu.async_copy(tab.at[idx_r], bufs[k], gsem[k])
    for k in range(nt):
        b = k % nb
        gathers[k].wait()
        writes[k] = pltpu.async_copy(bufs[b], tasks[k][2], wsem[b])
        nxt = k + nb
        if nxt < nt:
            writes[nxt - nb].wait()
            idx_r, tab, _ = tasks[nxt]
            gathers[nxt] = pltpu.async_copy(tab.at[idx_r], bufs[b], gsem[b])
    for k in range(max(nt - nb, 0), nt):
        writes[k].wait()


@functools.cache
def _sc_soc_fn():
  @functools.partial(
      pl.kernel,
      mesh=plsc.VectorSubcoreMesh(core_axis_name="c", subcore_axis_name="s"),
      out_type=[
          jax.ShapeDtypeStruct((B3, D), jnp.float32),
          jax.ShapeDtypeStruct((B3, D), jnp.float32),
      ],
      scratch_types=(
          [pltpu.VMEM((NCH, CHUNK), jnp.int32)] +
          [pltpu.VMEM((CHUNK, D), jnp.float32)] * NBUF +
          [pltpu.SemaphoreType.DMA] * (2 * NBUF)
      ),
  )
  def _sc_soc(soc_idx, entity_t, user_t, user3_o, entu3_o,
              idx_s, *bufs_sems):
    bufs = bufs_sems[:NBUF]
    gsem = bufs_sems[NBUF:2 * NBUF]
    wsem = bufs_sems[2 * NBUF:]
    wid = lax.axis_index("s") * 2 + lax.axis_index("c")
    row0 = wid * ROWS_W
    pltpu.sync_copy(soc_idx.at[wid], idx_s)
    tasks = (
        [(idx_s.at[j], user_t, user3_o.at[pl.ds(row0 + j * CHUNK, CHUNK)])
         for j in range(NCH)] +
        [(idx_s.at[j], entity_t, entu3_o.at[pl.ds(row0 + j * CHUNK, CHUNK)])
         for j in range(NCH)]
    )
    _ring_gather(tasks, bufs, gsem, wsem)

  return _sc_soc


@functools.cache
def _sc_ent_fn():
  @functools.partial(
      pl.kernel,
      mesh=plsc.VectorSubcoreMesh(core_axis_name="c", subcore_axis_name="s"),
      out_type=jax.ShapeDtypeStruct((B3, D), jnp.float32),
      scratch_types=(
          [pltpu.VMEM((NCH, CHUNK), jnp.int32)] +
          [pltpu.VMEM((CHUNK, D), jnp.float32)] * NCH +
          [pltpu.SemaphoreType.DMA] * (2 * NCH)
      ),
  )
  def _sc_ent(ent_idx, entity_t, ent3_o, idx_e, *bufs_sems):
    bufs = bufs_sems[:NCH]
    gsem = bufs_sems[NCH:2 * NCH]
    wsem = bufs_sems[2 * NCH:]
    wid = lax.axis_index("s") * 2 + lax.axis_index("c")
    row0 = wid * ROWS_W
    pltpu.sync_copy(ent_idx.at[wid], idx_e)
    tasks = [
        (idx_e.at[j], entity_t, ent3_o.at[pl.ds(row0 + j * CHUNK, CHUNK)])
        for j in range(NCH)
    ]
    _ring_gather(tasks, bufs, gsem, wsem)

  return _sc_ent


def _row_sums(x):
    return jnp.dot(x, jnp.ones((x.shape[1], 1), jnp.float32),
                   preferred_element_type=jnp.float32)


def _tc_body(lohi_ref, e3_ref, tm_ref, rs_ref, rsf_ref, relt_ref,
             u3_ref, eu3_ref, out_ref, acc_ref):
    i = pl.program_id(0)

    e_blk = e3_ref[...].reshape(3 * SM, D).astype(jnp.bfloat16)
    rr = rs_ref[...]
    lo = lohi_ref[0, i]
    hi = lohi_ref[1, i]

    for sec in range(3):
        acc_ref[pl.ds(sec * B + i * SM, SM), :] = jnp.zeros(
            (SM, K), jnp.float32)

    def trip(j, carry):
        mj = tm_ref[pl.ds(j * D, D), :]
        y = jnp.dot(e_blk, mj, preferred_element_type=jnp.float32)
        for sec in range(3):
            ys = y[sec * SM:(sec + 1) * SM]
            contrib = jnp.zeros((SM, K), jnp.float32)
            for q in range(RG):
                m = (rr == j * RG + q).astype(jnp.float32)
                contrib = contrib + ys[:, q * K:(q + 1) * K] * m
            acc_ref[pl.ds(sec * B + i * SM, SM), :] += contrib
        return carry

    lax.fori_loop(lo, hi + 1, trip, 0)

    @pl.when(i == NSB - 1)
    def _():
        def norm(x):
            n = jnp.sqrt(_row_sums(x * x))
            inv = 1.0 / jnp.maximum(n, 1e-12)
            return x * inv, n * inv

        rsf = rsf_ref[...]
        oh = (rsf == lax.broadcasted_iota(jnp.int32, (B, N_REL), 1)
              ).astype(jnp.float32)
        relg = jnp.dot(oh, relt_ref[...],
                       preferred_element_type=jnp.float32)

        h_n, h_u = norm(acc_ref[pl.ds(0, B), :])
        p_n, p_u = norm(acc_ref[pl.ds(B, B), :])
        n_n, n_u = norm(acc_ref[pl.ds(2 * B, B), :])
        r_n, r_u = norm(relg)
        base = h_n + r_n
        pos_score = _row_sums(jnp.square(base - p_n))
        neg_score = _row_sums(jnp.square(base - n_n))
        x = neg_score - pos_score
        kg_loss = jnp.mean(jnp.maximum(-x, 0.0)
                           + jnp.log(1.0 + jnp.exp(-jnp.abs(x))))
        kg_l2 = 0.5 * (jnp.mean(jnp.square(h_u)) + jnp.mean(jnp.square(r_u))
                       + jnp.mean(jnp.square(p_u))
                       + jnp.mean(jnp.square(n_u)))

        inv_u = u3_ref[pl.ds(0, B), :]
        inv_s = inv_u + eu3_ref[pl.ds(0, B), :]
        vp_s = u3_ref[pl.ds(B, B), :] + eu3_ref[pl.ds(B, B), :]
        vn_s = u3_ref[pl.ds(2 * B, B), :] + eu3_ref[pl.ds(2 * B, B), :]
        pos_s = _row_sums(inv_s * vp_s)
        neg_s = _row_sums(inv_s * vn_s)
        xs = pos_s - neg_s
        sig = 1.0 / (1.0 + jnp.exp(-xs))
        social_loss = jnp.mean(-jnp.log(1e-10 + sig))
        social_l2 = 0.5 * (jnp.mean(_row_sums(inv_u * inv_u))
                           + jnp.mean(_row_sums(vp_s * vp_s))
                           + jnp.mean(_row_sums(vn_s * vn_s)))

        out_ref[0, 0] = (kg_loss + KG_L2 * kg_l2
                         + social_loss + SOCIAL_L2 * social_l2)


def _tc_call(lohi, e3s, tm, rs, relt, u3, eu3):
    return pl.pallas_call(
        _tc_body,
        grid=(NSB,),
        in_specs=[
            pl.BlockSpec(memory_space=pltpu.SMEM),
            pl.BlockSpec((3, SM, D), lambda i: (0, i, 0)),
            pl.BlockSpec((NG * D, RG * K), lambda i: (0, 0)),
            pl.BlockSpec((SM, 1), lambda i: (i, 0)),
            pl.BlockSpec((B, 1), lambda i: (0, 0)),
            pl.BlockSpec((N_REL, K), lambda i: (0, 0)),
            pl.BlockSpec((B3, D), lambda i: (0, 0)),
            pl.BlockSpec((B3, D), lambda i: (0, 0)),
        ],
        out_specs=pl.BlockSpec(memory_space=pltpu.SMEM),
        out_shape=jax.ShapeDtypeStruct((1, 1), jnp.float32),
        scratch_shapes=[pltpu.VMEM((B3, K), jnp.float32)],
        compiler_params=pltpu.CompilerParams(
            dimension_semantics=("arbitrary",)),
    )(lohi, e3s, tm, rs, rs.reshape(B, 1), relt, u3, eu3)


def kernel(inviter_ids, voter_pos_ids, voter_neg_ids, h, r, pos_t, neg_t,
           is_train, user_embed, entity_embed, relation_embed, trans_M):
    i32 = jnp.int32
    soc_idx = jnp.concatenate(
        [inviter_ids, voter_pos_ids, voter_neg_ids]).astype(i32).reshape(
        NW, NCH, CHUNK)
    user3, entu3 = _sc_soc_fn()(soc_idx, entity_embed, user_embed)

    r_s, h_s, p_s, n_s = lax.sort(
        (r.astype(i32), h.astype(i32), pos_t.astype(i32), neg_t.astype(i32)),
        num_keys=1)
    ent_idx = jnp.concatenate([h_s, p_s, n_s]).reshape(NW, NCH, CHUNK)
    ent3 = _sc_ent_fn()(ent_idx, entity_embed)

    g_all = r_s // RG
    lohi = jnp.stack([g_all[::SM], g_all[SM - 1::SM]]).astype(i32)

    tm = trans_M.reshape(NG, RG, D, K).transpose(0, 2, 1, 3).reshape(
        NG * D, RG * K).astype(jnp.bfloat16)

    out = _tc_call(lohi, ent3.reshape(3, B, D), tm,
                   r_s.reshape(B, 1), relation_embed, user3, entu3)
    return out.reshape(())


# speedup vs baseline: 1.0566x; 1.0434x over previous
import functools

import jax
import jax.numpy as jnp
from jax import lax
from jax.experimental import pallas as pl
from jax.experimental.pallas import tpu as pltpu
from jax.experimental.pallas import tpu_sc as plsc

B = 4096
B3 = 3 * B
D = 128
K = 64
N_REL = 64
KG_L2 = 1e-05
SOCIAL_L2 = 1e-05

NW = 32
CHUNK = 128
NCH = B3 // NW // CHUNK
ROWS_W = B3 // NW
NBUF = 6

SM = 256
NSB = B // SM
RG = 4
NG = N_REL // RG


def _ring_gather(tasks, bufs, gsem, wsem):
    nb = len(bufs)
    nt = len(tasks)
    gathers = [None] * nt
    writes = [None] * nt
    for k in range(min(nb, nt)):
        idx_r, tab, _ = tasks[k]
        gathers[k] = pltpu.async_copy(tab.at[idx_r], bufs[k], gsem[k])
    for k in range(nt):
        b = k % nb
        gathers[k].wait()
        writes[k] = pltpu.async_copy(bufs[b], tasks[k][2], wsem[b])
        nxt = k + nb
        if nxt < nt:
            writes[nxt - nb].wait()
            idx_r, tab, _ = tasks[nxt]
            gathers[nxt] = pltpu.async_copy(tab.at[idx_r], bufs[b], gsem[b])
    for k in range(max(nt - nb, 0), nt):
        writes[k].wait()


@functools.cache
def _sc_gather_fn():
  @functools.partial(
      pl.kernel,
      mesh=plsc.VectorSubcoreMesh(core_axis_name="c", subcore_axis_name="s"),
      out_type=[
          jax.ShapeDtypeStruct((B3, D), jnp.float32),
          jax.ShapeDtypeStruct((B3, D), jnp.float32),
          jax.ShapeDtypeStruct((B3, D), jnp.float32),
      ],
      scratch_types=(
          [pltpu.VMEM((NCH, CHUNK), jnp.int32)] * 2 +
          [pltpu.VMEM((CHUNK, D), jnp.float32)] * NBUF +
          [pltpu.SemaphoreType.DMA] * (2 * NBUF)
      ),
  )
  def _sc_gather(ent_idx, soc_idx, entity_t, user_t,
                 ent3_o, user3_o, entu3_o,
                 idx_e, idx_s, *bufs_sems):
    bufs = bufs_sems[:NBUF]
    gsem = bufs_sems[NBUF:2 * NBUF]
    wsem = bufs_sems[2 * NBUF:]

    wid = lax.axis_index("s") * 2 + lax.axis_index("c")
    row0 = wid * ROWS_W

    pltpu.sync_copy(ent_idx.at[wid], idx_e)
    pltpu.sync_copy(soc_idx.at[wid], idx_s)

    tasks = (
        [(idx_e.at[j], entity_t, ent3_o.at[pl.ds(row0 + j * CHUNK, CHUNK)])
         for j in range(NCH)] +
        [(idx_s.at[j], user_t, user3_o.at[pl.ds(row0 + j * CHUNK, CHUNK)])
         for j in range(NCH)] +
        [(idx_s.at[j], entity_t, entu3_o.at[pl.ds(row0 + j * CHUNK, CHUNK)])
         for j in range(NCH)]
    )
    _ring_gather(tasks, bufs, gsem, wsem)

  return _sc_gather


def _row_sums(x):
    return jnp.dot(x, jnp.ones((x.shape[1], 1), jnp.float32),
                   preferred_element_type=jnp.float32)


def _tc_body(lohi_ref, e3_ref, tm_ref, rs_ref, rsf_ref, relt_ref,
             u3_ref, eu3_ref, out_ref, acc_ref):
    i = pl.program_id(0)

    e_blk = e3_ref[...].reshape(3 * SM, D).astype(jnp.bfloat16)
    rr = rs_ref[...]
    lo = lohi_ref[0, i]
    hi = lohi_ref[1, i]

    for sec in range(3):
        acc_ref[pl.ds(sec * B + i * SM, SM), :] = jnp.zeros(
            (SM, K), jnp.float32)

    def trip(j, carry):
        mj = tm_ref[pl.ds(j * D, D), :]
        y = jnp.dot(e_blk, mj, preferred_element_type=jnp.float32)
        for sec in range(3):
            ys = y[sec * SM:(sec + 1) * SM]
            contrib = jnp.zeros((SM, K), jnp.float32)
            for q in range(RG):
                m = (rr == j * RG + q).astype(jnp.float32)
                contrib = contrib + ys[:, q * K:(q + 1) * K] * m
            acc_ref[pl.ds(sec * B + i * SM, SM), :] += contrib
        return carry

    lax.fori_loop(lo, hi + 1, trip, 0)

    @pl.when(i == NSB - 1)
    def _():
        def norm(x):
            n = jnp.sqrt(_row_sums(x * x))
            inv = 1.0 / jnp.maximum(n, 1e-12)
            return x * inv, n * inv

        rsf = rsf_ref[...]
        oh = (rsf == lax.broadcasted_iota(jnp.int32, (B, N_REL), 1)
              ).astype(jnp.float32)
        relg = jnp.dot(oh, relt_ref[...],
                       preferred_element_type=jnp.float32)

        h_n, h_u = norm(acc_ref[pl.ds(0, B), :])
        p_n, p_u = norm(acc_ref[pl.ds(B, B), :])
        n_n, n_u = norm(acc_ref[pl.ds(2 * B, B), :])
        r_n, r_u = norm(relg)
        base = h_n + r_n
        pos_score = _row_sums(jnp.square(base - p_n))
        neg_score = _row_sums(jnp.square(base - n_n))
        x = neg_score - pos_score
        kg_loss = jnp.mean(jnp.maximum(-x, 0.0)
                           + jnp.log(1.0 + jnp.exp(-jnp.abs(x))))
        kg_l2 = 0.5 * (jnp.mean(jnp.square(h_u)) + jnp.mean(jnp.square(r_u))
                       + jnp.mean(jnp.square(p_u))
                       + jnp.mean(jnp.square(n_u)))

        inv_u = u3_ref[pl.ds(0, B), :]
        inv_s = inv_u + eu3_ref[pl.ds(0, B), :]
        vp_s = u3_ref[pl.ds(B, B), :] + eu3_ref[pl.ds(B, B), :]
        vn_s = u3_ref[pl.ds(2 * B, B), :] + eu3_ref[pl.ds(2 * B, B), :]
        pos_s = _row_sums(inv_s * vp_s)
        neg_s = _row_sums(inv_s * vn_s)
        xs = pos_s - neg_s
        sig = 1.0 / (1.0 + jnp.exp(-xs))
        social_loss = jnp.mean(-jnp.log(1e-10 + sig))
        social_l2 = 0.5 * (jnp.mean(_row_sums(inv_u * inv_u))
                           + jnp.mean(_row_sums(vp_s * vp_s))
                           + jnp.mean(_row_sums(vn_s * vn_s)))

        out_ref[0, 0] = (kg_loss + KG_L2 * kg_l2
                         + social_loss + SOCIAL_L2 * social_l2)


def _tc_call(lohi, e3s, tm, rs, relt, u3, eu3):
    return pl.pallas_call(
        _tc_body,
        grid=(NSB,),
        in_specs=[
            pl.BlockSpec(memory_space=pltpu.SMEM),
            pl.BlockSpec((3, SM, D), lambda i: (0, i, 0)),
            pl.BlockSpec((NG * D, RG * K), lambda i: (0, 0)),
            pl.BlockSpec((SM, 1), lambda i: (i, 0)),
            pl.BlockSpec((B, 1), lambda i: (0, 0)),
            pl.BlockSpec((N_REL, K), lambda i: (0, 0)),
            pl.BlockSpec((B3, D), lambda i: (0, 0)),
            pl.BlockSpec((B3, D), lambda i: (0, 0)),
        ],
        out_specs=pl.BlockSpec(memory_space=pltpu.SMEM),
        out_shape=jax.ShapeDtypeStruct((1, 1), jnp.float32),
        scratch_shapes=[pltpu.VMEM((B3, K), jnp.float32)],
        compiler_params=pltpu.CompilerParams(
            dimension_semantics=("arbitrary",)),
    )(lohi, e3s, tm, rs, rs.reshape(B, 1), relt, u3, eu3)


def kernel(inviter_ids, voter_pos_ids, voter_neg_ids, h, r, pos_t, neg_t,
           is_train, user_embed, entity_embed, relation_embed, trans_M):
    i32 = jnp.int32
    r_s, h_s, p_s, n_s = lax.sort(
        (r.astype(i32), h.astype(i32), pos_t.astype(i32), neg_t.astype(i32)),
        num_keys=1)

    ent_idx = jnp.concatenate([h_s, p_s, n_s]).reshape(NW, NCH, CHUNK)
    soc_idx = jnp.concatenate(
        [inviter_ids, voter_pos_ids, voter_neg_ids]).astype(i32).reshape(
        NW, NCH, CHUNK)

    ent3, user3, entu3 = _sc_gather_fn()(
        ent_idx, soc_idx, entity_embed, user_embed)

    g_all = r_s // RG
    lohi = jnp.stack([g_all[::SM], g_all[SM - 1::SM]]).astype(i32)

    tm = trans_M.reshape(NG, RG, D, K).transpose(0, 2, 1, 3).reshape(
        NG * D, RG * K).astype(jnp.bfloat16)

    out = _tc_call(lohi, ent3.reshape(3, B, D), tm,
                   r_s.reshape(B, 1), relation_embed, user3, entu3)
    return out.reshape(())


# speedup vs baseline: 1.0578x; 1.0011x over previous
import functools

import jax
import jax.numpy as jnp
from jax import lax
from jax.experimental import pallas as pl
from jax.experimental.pallas import tpu as pltpu
from jax.experimental.pallas import tpu_sc as plsc

B = 4096
B3 = 3 * B
D = 128
K = 64
N_REL = 64
KG_L2 = 1e-05
SOCIAL_L2 = 1e-05

NW = 32
CHUNK = 128
NCH = B3 // NW // CHUNK
ROWS_W = B3 // NW
NBUF = 6

SM = 256
NSB = B // SM
RG = 4
NG = N_REL // RG


def _ring_gather(tasks, bufs, gsem, wsem):
    nb = len(bufs)
    nt = len(tasks)
    gathers = [None] * nt
    writes = [None] * nt
    for k in range(min(nb, nt)):
        idx_r, tab, _ = tasks[k]
        gathers[k] = pltpu.async_copy(tab.at[idx_r], bufs[k], gsem[k])
    for k in range(nt):
        b = k % nb
        gathers[k].wait()
        writes[k] = pltpu.async_copy(bufs[b], tasks[k][2], wsem[b])
        nxt = k + nb
        if nxt < nt:
            writes[nxt - nb].wait()
            idx_r, tab, _ = tasks[nxt]
            gathers[nxt] = pltpu.async_copy(tab.at[idx_r], bufs[b], gsem[b])
    for k in range(max(nt - nb, 0), nt):
        writes[k].wait()


@functools.cache
def _sc_gather_fn():
  @functools.partial(
      pl.kernel,
      mesh=plsc.VectorSubcoreMesh(core_axis_name="c", subcore_axis_name="s"),
      out_type=[
          jax.ShapeDtypeStruct((B3, D), jnp.float32),
          jax.ShapeDtypeStruct((B3, D), jnp.float32),
          jax.ShapeDtypeStruct((B3, D), jnp.float32),
      ],
      scratch_types=(
          [pltpu.VMEM((NCH, CHUNK), jnp.int32)] * 2 +
          [pltpu.VMEM((CHUNK, D), jnp.float32)] * NBUF +
          [pltpu.SemaphoreType.DMA] * (2 * NBUF)
      ),
  )
  def _sc_gather(ent_idx, soc_idx, entity_t, user_t,
                 ent3_o, user3_o, entu3_o,
                 idx_e, idx_s, *bufs_sems):
    bufs = bufs_sems[:NBUF]
    gsem = bufs_sems[NBUF:2 * NBUF]
    wsem = bufs_sems[2 * NBUF:]

    wid = lax.axis_index("s") * 2 + lax.axis_index("c")
    row0 = wid * ROWS_W

    pltpu.sync_copy(ent_idx.at[wid], idx_e)
    pltpu.sync_copy(soc_idx.at[wid], idx_s)

    tasks = (
        [(idx_e.at[j], entity_t, ent3_o.at[pl.ds(row0 + j * CHUNK, CHUNK)])
         for j in range(NCH)] +
        [(idx_s.at[j], user_t, user3_o.at[pl.ds(row0 + j * CHUNK, CHUNK)])
         for j in range(NCH)] +
        [(idx_s.at[j], entity_t, entu3_o.at[pl.ds(row0 + j * CHUNK, CHUNK)])
         for j in range(NCH)]
    )
    _ring_gather(tasks, bufs, gsem, wsem)

  return _sc_gather


def _row_sums(x):
    return jnp.dot(x, jnp.ones((x.shape[1], 1), jnp.float32),
                   preferred_element_type=jnp.float32)


def _tc_body(lohi_ref, e3_ref, tm_ref, rs_ref, rsf_ref, relt_ref,
             u3_ref, eu3_ref, out_ref, acc_ref):
    i = pl.program_id(0)

    e_blk = e3_ref[...].reshape(3 * SM, D).astype(jnp.bfloat16)
    rr = rs_ref[...]
    lo = lohi_ref[0, i]
    hi = lohi_ref[1, i]

    blk = pl.ds(i * 3 * SM, 3 * SM)
    acc_ref[blk, :] = jnp.zeros((3 * SM, K), jnp.float32)

    def trip(j, carry):
        mj = tm_ref[pl.ds(j * D, D), :]
        y = jnp.dot(e_blk, mj, preferred_element_type=jnp.float32)
        contrib = jnp.zeros((3 * SM, K), jnp.float32)
        for q in range(RG):
            m = (rr == j * RG + q).astype(jnp.float32)
            m3 = jnp.concatenate([m, m, m])
            contrib = contrib + y[:, q * K:(q + 1) * K] * m3
        acc_ref[blk, :] += contrib
        return carry

    lax.fori_loop(lo, hi + 1, trip, 0)

    @pl.when(i == NSB - 1)
    def _():
        def norm(x):
            n = jnp.sqrt(_row_sums(x * x))
            inv = 1.0 / jnp.maximum(n, 1e-12)
            return x * inv, n * inv

        rsf = rsf_ref[...]
        oh = (rsf == lax.broadcasted_iota(jnp.int32, (B, N_REL), 1)
              ).astype(jnp.float32)
        relg = jnp.dot(oh, relt_ref[...],
                       preferred_element_type=jnp.float32)

        def section(sec):
            return jnp.concatenate([
                acc_ref[pl.ds((ib * 3 + sec) * SM, SM), :]
                for ib in range(NSB)])

        h_n, h_u = norm(section(0))
        p_n, p_u = norm(section(1))
        n_n, n_u = norm(section(2))
        r_n, r_u = norm(relg)
        base = h_n + r_n
        pos_score = _row_sums(jnp.square(base - p_n))
        neg_score = _row_sums(jnp.square(base - n_n))
        x = neg_score - pos_score
        kg_loss = jnp.mean(jnp.maximum(-x, 0.0)
                           + jnp.log(1.0 + jnp.exp(-jnp.abs(x))))
        kg_l2 = 0.5 * (jnp.mean(jnp.square(h_u)) + jnp.mean(jnp.square(r_u))
                       + jnp.mean(jnp.square(p_u))
                       + jnp.mean(jnp.square(n_u)))

        inv_u = u3_ref[pl.ds(0, B), :]
        inv_s = inv_u + eu3_ref[pl.ds(0, B), :]
        vp_s = u3_ref[pl.ds(B, B), :] + eu3_ref[pl.ds(B, B), :]
        vn_s = u3_ref[pl.ds(2 * B, B), :] + eu3_ref[pl.ds(2 * B, B), :]
        pos_s = _row_sums(inv_s * vp_s)
        neg_s = _row_sums(inv_s * vn_s)
        xs = pos_s - neg_s
        sig = 1.0 / (1.0 + jnp.exp(-xs))
        social_loss = jnp.mean(-jnp.log(1e-10 + sig))
        social_l2 = 0.5 * (jnp.mean(_row_sums(inv_u * inv_u))
                           + jnp.mean(_row_sums(vp_s * vp_s))
                           + jnp.mean(_row_sums(vn_s * vn_s)))

        out_ref[0, 0] = (kg_loss + KG_L2 * kg_l2
                         + social_loss + SOCIAL_L2 * social_l2)


def _tc_call(lohi, e3s, tm, rs, relt, u3, eu3):
    return pl.pallas_call(
        _tc_body,
        grid=(NSB,),
        in_specs=[
            pl.BlockSpec(memory_space=pltpu.SMEM),
            pl.BlockSpec((3, SM, D), lambda i: (0, i, 0)),
            pl.BlockSpec((NG * D, RG * K), lambda i: (0, 0)),
            pl.BlockSpec((SM, 1), lambda i: (i, 0)),
            pl.BlockSpec((B, 1), lambda i: (0, 0)),
            pl.BlockSpec((N_REL, K), lambda i: (0, 0)),
            pl.BlockSpec((B3, D), lambda i: (0, 0)),
            pl.BlockSpec((B3, D), lambda i: (0, 0)),
        ],
        out_specs=pl.BlockSpec(memory_space=pltpu.SMEM),
        out_shape=jax.ShapeDtypeStruct((1, 1), jnp.float32),
        scratch_shapes=[pltpu.VMEM((B3, K), jnp.float32)],
        compiler_params=pltpu.CompilerParams(
            dimension_semantics=("arbitrary",)),
    )(lohi, e3s, tm, rs, rs.reshape(B, 1), relt, u3, eu3)


def kernel(inviter_ids, voter_pos_ids, voter_neg_ids, h, r, pos_t, neg_t,
           is_train, user_embed, entity_embed, relation_embed, trans_M):
    i32 = jnp.int32
    r_s, h_s, p_s, n_s = lax.sort(
        (r.astype(i32), h.astype(i32), pos_t.astype(i32), neg_t.astype(i32)),
        num_keys=1)

    ent_idx = jnp.concatenate([h_s, p_s, n_s]).reshape(NW, NCH, CHUNK)
    soc_idx = jnp.concatenate(
        [inviter_ids, voter_pos_ids, voter_neg_ids]).astype(i32).reshape(
        NW, NCH, CHUNK)

    ent3, user3, entu3 = _sc_gather_fn()(
        ent_idx, soc_idx, entity_embed, user_embed)

    g_all = r_s // RG
    lohi = jnp.stack([g_all[::SM], g_all[SM - 1::SM]]).astype(i32)

    tm = trans_M.reshape(NG, RG, D, K).transpose(0, 2, 1, 3).reshape(
        NG * D, RG * K).astype(jnp.bfloat16)

    out = _tc_call(lohi, ent3.reshape(3, B, D), tm,
                   r_s.reshape(B, 1), relation_embed, user3, entu3)
    return out.reshape(())
